# baseline (speedup 1.0000x reference)
import functools
import math

import jax
import jax.numpy as jnp
import numpy as np
from jax import lax
from jax.experimental import pallas as pl
from jax.experimental.pallas import tpu as pltpu
from jax.experimental.pallas import tpu_sc as plsc

D_MODEL = 128
SEQ_LEN = 2048
BATCH = 4
SCALE = math.sqrt(float(D_MODEL))

NUM_CORES = 2
NUM_SUBCORES = 16
NUM_WORKERS = NUM_CORES * NUM_SUBCORES
POS_PER_W = SEQ_LEN // NUM_WORKERS
LANES = 16
CHUNKS = D_MODEL // LANES


def _positional_encoding() -> np.ndarray:
    half = D_MODEL / 2
    positions = np.arange(SEQ_LEN)[:, np.newaxis]
    depths = np.arange(int(half))[np.newaxis, :] / half
    angle_rates = 1 / 10000 ** depths
    angle_rads = positions * angle_rates
    pe = np.concatenate([np.sin(angle_rads), np.cos(angle_rads)], axis=-1)
    return pe.astype(np.float32)


_POS_ENC_FLAT = jnp.asarray(_positional_encoding().reshape(-1))


def _emb_body(x_hbm, pos_hbm, table_hbm, out_hbm, idx_v, pos_v, tile_v,
              gsem, psem, wsem):
    wid = lax.axis_index("s") * NUM_CORES + lax.axis_index("c")
    pstart = wid * POS_PER_W
    half = (wid % 2) * POS_PER_W

    pltpu.sync_copy(x_hbm.at[:, pl.ds((wid // 2) * 2 * POS_PER_W,
                                      2 * POS_PER_W)], idx_v)

    gcopies = []
    for b in range(BATCH):
        gcopies.append(pltpu.async_copy(
            table_hbm.at[idx_v.at[b, pl.ds(half, POS_PER_W)]],
            tile_v.at[pl.ds(b * POS_PER_W, POS_PER_W)],
            gsem))

    pcopy = pltpu.async_copy(
        pos_hbm.at[pl.ds(pstart * D_MODEL, POS_PER_W * D_MODEL)], pos_v,
        psem)
    pcopy.wait()

    ROWS_PER_IT = 4
    writes = []
    for b in range(BATCH):
        gcopies[b].wait()

        def row_body(i0, carry, b=b):
            for u in range(ROWS_PER_IT):
                i = i0 * ROWS_PER_IT + u
                r = b * POS_PER_W + i
                for c in range(CHUNKS):
                    tile_v[r, pl.ds(c * LANES, LANES)] = (
                        tile_v[r, pl.ds(c * LANES, LANES)] * SCALE
                        + pos_v[pl.ds(i * D_MODEL + c * LANES, LANES)])
            return carry

        lax.fori_loop(0, POS_PER_W // ROWS_PER_IT, row_body, 0)
        writes.append(pltpu.async_copy(
            tile_v.at[pl.ds(b * POS_PER_W, POS_PER_W)],
            out_hbm.at[b, pl.ds(pstart, POS_PER_W)],
            wsem))

    for w in writes:
        w.wait()


@jax.jit
def _emb_call(x_flat, pos_flat, table):
    mesh = plsc.VectorSubcoreMesh(core_axis_name="c", subcore_axis_name="s")
    run = functools.partial(
        pl.kernel,
        mesh=mesh,
        out_type=jax.ShapeDtypeStruct((BATCH, SEQ_LEN, D_MODEL), jnp.float32),
        scratch_types=[
            pltpu.VMEM((BATCH, 2 * POS_PER_W), jnp.int32),
            pltpu.VMEM((POS_PER_W * D_MODEL,), jnp.float32),
            pltpu.VMEM((BATCH * POS_PER_W, D_MODEL), jnp.float32),
            pltpu.SemaphoreType.DMA,
            pltpu.SemaphoreType.DMA,
            pltpu.SemaphoreType.DMA,
        ],
    )(_emb_body)
    return run(x_flat, pos_flat, table)


def kernel(x, table):
    return _emb_call(x, _POS_ENC_FLAT, table)


# speedup vs baseline: 1.0056x; 1.0056x over previous
import functools
import math

import jax
import jax.numpy as jnp
import numpy as np
from jax import lax
from jax.experimental import pallas as pl
from jax.experimental.pallas import tpu as pltpu
from jax.experimental.pallas import tpu_sc as plsc

D_MODEL = 128
SEQ_LEN = 2048
BATCH = 4
SCALE = math.sqrt(float(D_MODEL))

NUM_CORES = 2
NUM_SUBCORES = 16
NUM_WORKERS = NUM_CORES * NUM_SUBCORES
POS_PER_W = SEQ_LEN // NUM_WORKERS
LANES = 16
CHUNKS = D_MODEL // LANES


def _positional_encoding() -> np.ndarray:
    half = D_MODEL / 2
    positions = np.arange(SEQ_LEN)[:, np.newaxis]
    depths = np.arange(int(half))[np.newaxis, :] / half
    angle_rates = 1 / 10000 ** depths
    angle_rads = positions * angle_rates
    pe = np.concatenate([np.sin(angle_rads), np.cos(angle_rads)], axis=-1)
    return pe.astype(np.float32)


_POS_ENC_FLAT = jnp.asarray(_positional_encoding().reshape(-1))


def _emb_body(x_hbm, pos_hbm, table_hbm, out_hbm, idx_v, pos_v, tile_v,
              gsem, psem, wsem):
    wid = lax.axis_index("s") * NUM_CORES + lax.axis_index("c")
    pstart = wid * POS_PER_W
    half = (wid % 2) * POS_PER_W

    pltpu.sync_copy(x_hbm.at[:, pl.ds((wid // 2) * 2 * POS_PER_W,
                                      2 * POS_PER_W)], idx_v)

    gcopies = []
    for b in range(BATCH):
        gcopies.append(pltpu.async_copy(
            table_hbm.at[idx_v.at[b, pl.ds(half, POS_PER_W)]],
            tile_v.at[pl.ds(b * POS_PER_W, POS_PER_W)],
            gsem))

    pcopy = pltpu.async_copy(
        pos_hbm.at[pl.ds(pstart * D_MODEL, POS_PER_W * D_MODEL)], pos_v,
        psem)
    pcopy.wait()

    for b in range(BATCH):
        gcopies[b].wait()

    def row_body(i, carry):
        for c in range(CHUNKS):
            pv = pos_v[pl.ds(i * D_MODEL + c * LANES, LANES)]
            for b in range(BATCH):
                r = b * POS_PER_W + i
                tile_v[r, pl.ds(c * LANES, LANES)] = (
                    tile_v[r, pl.ds(c * LANES, LANES)] * SCALE + pv)
        return carry

    lax.fori_loop(0, POS_PER_W, row_body, 0)

    writes = []
    for b in range(BATCH):
        writes.append(pltpu.async_copy(
            tile_v.at[pl.ds(b * POS_PER_W, POS_PER_W)],
            out_hbm.at[b, pl.ds(pstart, POS_PER_W)],
            wsem))
    for w in writes:
        w.wait()


@jax.jit
def _emb_call(x_flat, pos_flat, table):
    mesh = plsc.VectorSubcoreMesh(core_axis_name="c", subcore_axis_name="s")
    run = functools.partial(
        pl.kernel,
        mesh=mesh,
        out_type=jax.ShapeDtypeStruct((BATCH, SEQ_LEN, D_MODEL), jnp.float32),
        scratch_types=[
            pltpu.VMEM((BATCH, 2 * POS_PER_W), jnp.int32),
            pltpu.VMEM((POS_PER_W * D_MODEL,), jnp.float32),
            pltpu.VMEM((BATCH * POS_PER_W, D_MODEL), jnp.float32),
            pltpu.SemaphoreType.DMA,
            pltpu.SemaphoreType.DMA,
            pltpu.SemaphoreType.DMA,
        ],
    )(_emb_body)
    return run(x_flat, pos_flat, table)


def kernel(x, table):
    return _emb_call(x, _POS_ENC_FLAT, table)


# speedup vs baseline: 1.0281x; 1.0223x over previous
import functools
import math

import jax
import jax.numpy as jnp
import numpy as np
from jax import lax
from jax.experimental import pallas as pl
from jax.experimental.pallas import tpu as pltpu
from jax.experimental.pallas import tpu_sc as plsc

D_MODEL = 128
SEQ_LEN = 2048
BATCH = 4
SCALE = math.sqrt(float(D_MODEL))

NUM_CORES = 2
NUM_SUBCORES = 16
NUM_WORKERS = NUM_CORES * NUM_SUBCORES
POS_PER_W = SEQ_LEN // NUM_WORKERS
LANES = 16
CHUNKS = D_MODEL // LANES


def _positional_encoding() -> np.ndarray:
    half = D_MODEL / 2
    positions = np.arange(SEQ_LEN)[:, np.newaxis]
    depths = np.arange(int(half))[np.newaxis, :] / half
    angle_rates = 1 / 10000 ** depths
    angle_rads = positions * angle_rates
    pe = np.concatenate([np.sin(angle_rads), np.cos(angle_rads)], axis=-1)
    return pe.astype(np.float32)


_POS_ENC_FLAT = jnp.asarray(_positional_encoding().reshape(-1))


def _emb_body(x_hbm, pos_hbm, table_hbm, out_hbm, idx_v, pos_v, tile_v,
              isem, gsem, psem, wsem):
    wid = lax.axis_index("s") * NUM_CORES + lax.axis_index("c")
    pstart = wid * POS_PER_W
    half = (wid % 2) * POS_PER_W
    win = (wid // 2) * 2 * POS_PER_W

    icopies = []
    for b in range(BATCH):
        icopies.append(pltpu.async_copy(
            x_hbm.at[pl.ds(b, 1), pl.ds(win, 2 * POS_PER_W)],
            idx_v.at[pl.ds(b, 1)],
            isem))

    gcopies = []
    for b in range(BATCH):
        icopies[b].wait()
        gcopies.append(pltpu.async_copy(
            table_hbm.at[idx_v.at[b, pl.ds(half, POS_PER_W)]],
            tile_v.at[pl.ds(b * POS_PER_W, POS_PER_W)],
            gsem))

    pcopy = pltpu.async_copy(
        pos_hbm.at[pl.ds(pstart * D_MODEL, POS_PER_W * D_MODEL)], pos_v,
        psem)
    pcopy.wait()

    writes = []
    for b in range(BATCH):
        gcopies[b].wait()

        def row_body(i, carry, b=b):
            r = b * POS_PER_W + i
            for c in range(CHUNKS):
                tile_v[r, pl.ds(c * LANES, LANES)] = (
                    tile_v[r, pl.ds(c * LANES, LANES)] * SCALE
                    + pos_v[pl.ds(i * D_MODEL + c * LANES, LANES)])
            return carry

        lax.fori_loop(0, POS_PER_W, row_body, 0)
        writes.append(pltpu.async_copy(
            tile_v.at[pl.ds(b * POS_PER_W, POS_PER_W)],
            out_hbm.at[b, pl.ds(pstart, POS_PER_W)],
            wsem))

    for w in writes:
        w.wait()


@jax.jit
def _emb_call(x_flat, pos_flat, table):
    mesh = plsc.VectorSubcoreMesh(core_axis_name="c", subcore_axis_name="s")
    run = functools.partial(
        pl.kernel,
        mesh=mesh,
        out_type=jax.ShapeDtypeStruct((BATCH, SEQ_LEN, D_MODEL), jnp.float32),
        scratch_types=[
            pltpu.VMEM((BATCH, 2 * POS_PER_W), jnp.int32),
            pltpu.VMEM((POS_PER_W * D_MODEL,), jnp.float32),
            pltpu.VMEM((BATCH * POS_PER_W, D_MODEL), jnp.float32),
            pltpu.SemaphoreType.DMA,
            pltpu.SemaphoreType.DMA,
            pltpu.SemaphoreType.DMA,
            pltpu.SemaphoreType.DMA,
        ],
    )(_emb_body)
    return run(x_flat, pos_flat, table)


def kernel(x, table):
    return _emb_call(x, _POS_ENC_FLAT, table)
